# Initial kernel scaffold; baseline (speedup 1.0000x reference)
import jax
import jax.numpy as jnp
from jax.experimental import pallas as pl


def kernel(x, edge_index, edge_weight, W0, b0, W1, b1):
    raise NotImplementedError("write your pallas kernel here")


# speedup vs baseline: 4.3370x; 4.3370x over previous
import functools

import jax
import jax.numpy as jnp
from jax import lax
from jax.experimental import pallas as pl
from jax.experimental.pallas import tpu as pltpu
from jax.experimental.pallas import tpu_sc as plsc

N_NODES = 10000
N_EDGES = 320000
DIM = 128

NC = 2
NS = 16
NW = NC * NS
CHUNK = 128
E_PAD = ((N_EDGES + NW * CHUNK - 1) // (NW * CHUNK)) * (NW * CHUNK)
EPT = E_PAD // NW
NCHUNK = EPT // CHUNK
N_PAD = 10240
RPT = N_PAD // NS


def _spmm_body(h_hbm, src_hbm, dst_hbm, w_hbm, zeros_hbm, out_hbm,
               src_v, dst_v, w_v, rows_v, acc_sh, sem):
    c = lax.axis_index("c")
    s = lax.axis_index("s")
    wid = c * NS + s

    pltpu.sync_copy(zeros_hbm, acc_sh.at[pl.ds(s * RPT, RPT)])
    pltpu.sync_copy(src_hbm.at[wid], src_v)
    pltpu.sync_copy(dst_hbm.at[wid], dst_v)
    pltpu.sync_copy(w_hbm.at[wid], w_v)
    plsc.subcore_barrier()

    def chunk_body(j, carry):
        pltpu.async_copy(h_hbm.at[src_v.at[j]], rows_v, sem).wait()

        def group_body(g, carry2):
            wv16 = w_v[pl.ds(j * CHUNK + g * 16, 16)]
            for t in range(16):
                e = g * 16 + t
                ws = wv16[t]
                for k in range(DIM // 16):
                    sl = pl.ds(k * 16, 16)
                    rows_v[e, sl] = rows_v[e, sl] * ws
            return carry2

        lax.fori_loop(0, CHUNK // 16, group_body, 0)
        pltpu.sync_copy(rows_v, acc_sh.at[dst_v.at[j]], add=True)
        return carry

    lax.fori_loop(0, NCHUNK, chunk_body, 0)
    plsc.subcore_barrier()
    pltpu.sync_copy(acc_sh.at[pl.ds(s * RPT, RPT)],
                    out_hbm.at[c, pl.ds(s * RPT, RPT)])


_spmm_sc = functools.partial(
    pl.kernel,
    out_type=jax.ShapeDtypeStruct((NC, N_PAD, DIM), jnp.float32),
    mesh=plsc.VectorSubcoreMesh(core_axis_name="c", subcore_axis_name="s"),
    scratch_types=[
        pltpu.VMEM((NCHUNK, CHUNK), jnp.int32),
        pltpu.VMEM((NCHUNK, CHUNK), jnp.int32),
        pltpu.VMEM((NCHUNK * CHUNK,), jnp.float32),
        pltpu.VMEM((CHUNK, DIM), jnp.float32),
        pltpu.VMEM_SHARED((N_PAD, DIM), jnp.float32),
        pltpu.SemaphoreType.DMA,
    ],
)(_spmm_body)


_BLK = 2000


def _li0_tc(x_ref, w_ref, b_ref, o_ref):
    o_ref[...] = (
        jnp.dot(x_ref[...], w_ref[...], preferred_element_type=jnp.float32)
        + b_ref[...])


def _add_tc(a_ref, b_ref, o_ref):
    o_ref[...] = a_ref[...] + b_ref[...]


def _li1_tc(a_ref, b_ref, w_ref, bias_ref, o_ref):
    h = jnp.maximum(a_ref[...] + b_ref[...], 0.0)
    o_ref[...] = (
        jnp.dot(h, w_ref[...], preferred_element_type=jnp.float32)
        + bias_ref[...])


def _row_spec():
    return pl.BlockSpec((_BLK, DIM), lambda i: (i, 0))


def _full_spec(shape):
    return pl.BlockSpec(shape, lambda i: (0,) * len(shape))


def _li0(x, W0, b0):
    return pl.pallas_call(
        _li0_tc,
        grid=(N_NODES // _BLK,),
        in_specs=[_row_spec(), _full_spec((DIM, DIM)), _full_spec((1, DIM))],
        out_specs=_row_spec(),
        out_shape=jax.ShapeDtypeStruct((N_NODES, DIM), jnp.float32),
    )(x, W0, b0.reshape(1, DIM))


def _add(p):
    return pl.pallas_call(
        _add_tc,
        grid=(N_NODES // _BLK,),
        in_specs=[_row_spec(), _row_spec()],
        out_specs=_row_spec(),
        out_shape=jax.ShapeDtypeStruct((N_NODES, DIM), jnp.float32),
    )(p[0], p[1])


def _li1(q, W1, b1):
    return pl.pallas_call(
        _li1_tc,
        grid=(N_NODES // _BLK,),
        in_specs=[_row_spec(), _row_spec(), _full_spec((DIM, DIM)),
                  _full_spec((1, DIM))],
        out_specs=_row_spec(),
        out_shape=jax.ShapeDtypeStruct((N_NODES, DIM), jnp.float32),
    )(q[0], q[1], W1, b1.reshape(1, DIM))


def kernel(x, edge_index, edge_weight, W0, b0, W1, b1):
    pad = E_PAD - N_EDGES
    src = jnp.pad(edge_index[0].astype(jnp.int32), (0, pad)).reshape(
        NW, NCHUNK, CHUNK)
    dst = jnp.pad(edge_index[1].astype(jnp.int32), (0, pad)).reshape(
        NW, NCHUNK, CHUNK)
    w = jnp.pad(edge_weight.astype(jnp.float32), (0, pad)).reshape(
        NW, NCHUNK * CHUNK)
    zeros = jnp.zeros((RPT, DIM), jnp.float32)

    h0 = _li0(x, W0, b0)
    p = _spmm_sc(h0, src, dst, w, zeros)
    h1 = _add(p)
    q = _spmm_sc(h1, src, dst, w, zeros)
    return _li1(q, W1, b1)
